# Initial kernel scaffold; baseline (speedup 1.0000x reference)
import jax
import jax.numpy as jnp
from jax.experimental import pallas as pl


def kernel(centers, contexts, neg_contexts, item_embed, context_embed):
    raise NotImplementedError("write your pallas kernel here")


# speedup vs baseline: 4.8690x; 4.8690x over previous
import functools

import jax
import jax.numpy as jnp
import numpy as np
from jax import lax
from jax.experimental import pallas as pl
from jax.experimental.pallas import tpu as pltpu
from jax.experimental.pallas import tpu_sc as plsc

B = 16384
K = 20
D = 64
KP1 = K + 1
P = B * KP1

NC = 2
NS = 16
NW = NC * NS

BW = B // NW
CB = 32
NCHUNK = BW // CB
PC = CB * KP1
GSZ = 112
G = PC // GSZ

_SIGN = np.where(np.arange(PC) % KP1 == 0, -1.0, 1.0).astype(np.float32)


def _sc_logits(centers, cat, item_embed, context_embed, sign):
  mesh = plsc.VectorSubcoreMesh(core_axis_name="c", subcore_axis_name="s")

  @functools.partial(
      pl.kernel,
      out_type=jax.ShapeDtypeStruct((P,), jnp.float32),
      mesh=mesh,
      compiler_params=pltpu.CompilerParams(
          needs_layout_passes=False, use_tc_tiling_on_sc=False),
      scratch_types=[
          pltpu.VMEM((CB,), jnp.int32),
          pltpu.VMEM((PC,), jnp.int32),
          pltpu.VMEM((CB, D), jnp.float32),
          pltpu.VMEM((PC, D), jnp.float32),
          pltpu.VMEM((PC,), jnp.float32),
          pltpu.VMEM((PC,), jnp.float32),
          pltpu.SemaphoreType.DMA,
      ],
  )
  def body(centers_hbm, cat_hbm, item_hbm, ctx_hbm, sign_hbm, out_hbm,
           cidx_v, catidx_v, crow_v, xrow_v, out_v, sign_v, sem):
    wid = lax.axis_index("s") * NC + lax.axis_index("c")
    pltpu.sync_copy(sign_hbm, sign_v)

    def chunk_body(cb, carry):
      cbase = wid * BW + cb * CB
      pltpu.sync_copy(centers_hbm.at[pl.ds(cbase, CB)], cidx_v)
      pltpu.sync_copy(cat_hbm.at[pl.ds(cbase * KP1, PC)], catidx_v)
      pltpu.async_copy(item_hbm.at[cidx_v], crow_v, sem).wait()
      for g in range(G):
        pltpu.async_copy(
            ctx_hbm.at[catidx_v.at[pl.ds(g * GSZ, GSZ)]],
            xrow_v.at[pl.ds(g * GSZ, GSZ)],
            sem,
        ).wait()

      lanes = lax.iota(jnp.int32, 16)

      def grp_body(g, c2):
        sg = sign_v[pl.ds(g * 16, 16)]
        merged = jnp.zeros((16,), jnp.float32)
        for l in range(16):
          p = g * 16 + l
          b = p // KP1
          acc = (crow_v[b, pl.ds(0, 16)] * xrow_v[p, pl.ds(0, 16)]
                 + crow_v[b, pl.ds(16, 16)] * xrow_v[p, pl.ds(16, 16)]
                 + crow_v[b, pl.ds(32, 16)] * xrow_v[p, pl.ds(32, 16)]
                 + crow_v[b, pl.ds(48, 16)] * xrow_v[p, pl.ds(48, 16)])
          s = jnp.sum(acc)
          merged = jnp.where(lanes == l, s, merged)
        out_v[pl.ds(g * 16, 16)] = merged * sg
        return c2

      lax.fori_loop(0, PC // 16, grp_body, 0)
      pltpu.sync_copy(out_v, out_hbm.at[pl.ds(cbase * KP1, PC)])
      return carry

    lax.fori_loop(0, NCHUNK, chunk_body, 0)

  return body(centers, cat, item_embed, context_embed, sign)


def _tc_loss(logits2d):
  def body(x_ref, o_ref):
    x = x_ref[...]
    o_ref[0, 0] = jnp.sum(jnp.log(1.0 + jnp.exp(x)))

  out = pl.pallas_call(
      body,
      out_shape=jax.ShapeDtypeStruct((1, 1), jnp.float32),
      out_specs=pl.BlockSpec(memory_space=pltpu.SMEM),
  )(logits2d)
  return out[0, 0] / B


def kernel(centers, contexts, neg_contexts, item_embed, context_embed):
  centers = centers.astype(jnp.int32)
  cat = jnp.concatenate(
      [contexts[:, None], neg_contexts], axis=1).astype(jnp.int32)
  logits = _sc_logits(centers, cat.reshape(P), item_embed, context_embed,
                      jnp.asarray(_SIGN))
  return _tc_loss(logits.reshape(P // 128, 128))


# speedup vs baseline: 5.2538x; 1.0790x over previous
import functools

import jax
import jax.numpy as jnp
import numpy as np
from jax import lax
from jax.experimental import pallas as pl
from jax.experimental.pallas import tpu as pltpu
from jax.experimental.pallas import tpu_sc as plsc

B = 16384
K = 20
D = 64
KP1 = K + 1
P = B * KP1

NC = 2
NS = 16
NW = NC * NS

BW = B // NW
CB = 32
NCHUNK = BW // CB
PC = CB * KP1
IC = CB + PC
GSZ = 112
G = PC // GSZ

_SIGN = np.where(np.arange(PC) % KP1 == 0, -1.0, 1.0).astype(np.float32)


def _sc_logits(idx, item_embed, context_embed, sign):
  mesh = plsc.VectorSubcoreMesh(core_axis_name="c", subcore_axis_name="s")

  @functools.partial(
      pl.kernel,
      out_type=jax.ShapeDtypeStruct((P,), jnp.float32),
      mesh=mesh,
      compiler_params=pltpu.CompilerParams(
          needs_layout_passes=False, use_tc_tiling_on_sc=False),
      scratch_types=[
          pltpu.VMEM((2, IC), jnp.int32),
          pltpu.VMEM((2, CB, D), jnp.float32),
          pltpu.VMEM((2, PC, D), jnp.float32),
          pltpu.VMEM((2, PC), jnp.float32),
          pltpu.VMEM((PC,), jnp.float32),
          pltpu.SemaphoreType.DMA,
          pltpu.SemaphoreType.DMA,
          pltpu.SemaphoreType.DMA,
      ],
  )
  def body(idx_hbm, item_hbm, ctx_hbm, sign_hbm, out_hbm,
           idx_v, crow_v, xrow_v, out_v, sign_v, sem0, sem1, osem):
    wid = lax.axis_index("s") * NC + lax.axis_index("c")
    pltpu.sync_copy(sign_hbm, sign_v)
    sems = (sem0, sem1)

    def fire(cb, slot):
      pltpu.sync_copy(idx_hbm.at[pl.ds((wid * NCHUNK + cb) * IC, IC)],
                      idx_v.at[slot])
      sem = sems[slot]
      hs = [pltpu.async_copy(item_hbm.at[idx_v.at[slot, pl.ds(0, CB)]],
                             crow_v.at[slot], sem)]
      for g in range(G):
        hs.append(pltpu.async_copy(
            ctx_hbm.at[idx_v.at[slot, pl.ds(CB + g * GSZ, GSZ)]],
            xrow_v.at[slot, pl.ds(g * GSZ, GSZ)],
            sem,
        ))
      return hs

    lanes = lax.iota(jnp.int32, 16)

    def compute(cb, slot):
      def grp_body(g, c2):
        sg = sign_v[pl.ds(g * 16, 16)]
        merged = jnp.zeros((16,), jnp.float32)
        for l in range(16):
          p = g * 16 + l
          b = p // KP1
          acc = (crow_v[slot, b, pl.ds(0, 16)]
                 * xrow_v[slot, p, pl.ds(0, 16)]
                 + crow_v[slot, b, pl.ds(16, 16)]
                 * xrow_v[slot, p, pl.ds(16, 16)]
                 + crow_v[slot, b, pl.ds(32, 16)]
                 * xrow_v[slot, p, pl.ds(32, 16)]
                 + crow_v[slot, b, pl.ds(48, 16)]
                 * xrow_v[slot, p, pl.ds(48, 16)])
          s = jnp.sum(acc)
          merged = jnp.where(lanes == l, s, merged)
        out_v[slot, pl.ds(g * 16, 16)] = merged * sg
        return c2

      lax.fori_loop(0, PC // 16, grp_body, 0, unroll=2)
      return pltpu.async_copy(
          out_v.at[slot],
          out_hbm.at[pl.ds((wid * BW + cb * CB) * KP1, PC)],
          osem)

    inflight = fire(0, 0)
    out_h = [None, None]
    for cb in range(NCHUNK):
      slot = cb % 2
      nxt = fire(cb + 1, 1 - slot) if cb + 1 < NCHUNK else []
      for h in inflight:
        h.wait()
      if out_h[slot] is not None:
        out_h[slot].wait()
      out_h[slot] = compute(cb, slot)
      inflight = nxt
    for h in out_h:
      if h is not None:
        h.wait()

  return body(idx, item_embed, context_embed, sign)


def _tc_loss(logits2d):
  def body(x_ref, o_ref):
    x = x_ref[...]
    o_ref[0, 0] = jnp.sum(jnp.log(1.0 + jnp.exp(x)))

  out = pl.pallas_call(
      body,
      out_shape=jax.ShapeDtypeStruct((1, 1), jnp.float32),
      out_specs=pl.BlockSpec(memory_space=pltpu.SMEM),
  )(logits2d)
  return out[0, 0] / B


def kernel(centers, contexts, neg_contexts, item_embed, context_embed):
  cat = jnp.concatenate(
      [contexts[:, None], neg_contexts], axis=1).astype(jnp.int32)
  idx = jnp.concatenate(
      [centers.astype(jnp.int32).reshape(B // CB, CB),
       cat.reshape(B // CB, PC)], axis=1).reshape(-1)
  logits = _sc_logits(idx, item_embed, context_embed, jnp.asarray(_SIGN))
  return _tc_loss(logits.reshape(P // 128, 128))
